# Initial kernel scaffold; baseline (speedup 1.0000x reference)
import jax
import jax.numpy as jnp
from jax.experimental import pallas as pl


def kernel(z, codebook, W1, b1, W2, b2):
    raise NotImplementedError("write your pallas kernel here")


# speedup vs baseline: 4.5689x; 4.5689x over previous
import functools

import jax
import jax.numpy as jnp
import numpy as np
from jax.experimental import pallas as pl
from jax.experimental.pallas import tpu as pltpu

B, T, D = 32, 576, 256
N_SUB, K, SUB = 8, 256, 32
HIDDEN = 1024
CLASSES = 1000
CPAD = 1024
TM = 512


def _body(z_ref, cbT_ref, csq_ref, cbs_ref, w1_ref, b1_ref, w2_ref, b2_ref,
          out_ref, onehot_scr):
    zt = z_ref[...]
    for n in range(N_SUB):
        zn = zt[:, n * SUB:(n + 1) * SUB]
        dots = jnp.dot(zn, cbT_ref[n * SUB:(n + 1) * SUB, :])
        dist = csq_ref[n, :][None, :] - 2.0 * dots
        code = jnp.argmin(dist, axis=1)
        iota = jax.lax.broadcasted_iota(jnp.int32, (TM, K), 1)
        onehot_scr[:, n * K:(n + 1) * K] = (
            iota == code[:, None]).astype(jnp.float32)
    q = jnp.dot(onehot_scr[...], cbs_ref[...])
    h = jnp.maximum(jnp.dot(q, w1_ref[...]) + b1_ref[...], 0.0)
    out_ref[...] = jnp.dot(h, w2_ref[...]) + b2_ref[...]


@jax.jit
def kernel(z, codebook, W1, b1, W2, b2):
    b, t, d = z.shape
    n_sub, k, sub = codebook.shape
    bt = b * t
    z2 = z.reshape(bt, d)
    cbT = codebook.transpose(0, 2, 1).reshape(n_sub * sub, k)
    c_sq = jnp.sum(codebook * codebook, axis=-1)
    cbs = jnp.concatenate(
        [jnp.pad(codebook[n], ((0, 0), (n * sub, d - (n + 1) * sub)))
         for n in range(n_sub)], axis=0)
    W2p = jnp.pad(W2, ((0, 0), (0, CPAD - CLASSES)))
    b2p = jnp.pad(b2, (0, CPAD - CLASSES)).reshape(1, CPAD)
    b1r = b1.reshape(1, HIDDEN)

    grid = (bt // TM,)
    out = pl.pallas_call(
        _body,
        grid=grid,
        in_specs=[
            pl.BlockSpec((TM, d), lambda i: (i, 0)),
            pl.BlockSpec((n_sub * sub, k), lambda i: (0, 0)),
            pl.BlockSpec((n_sub, k), lambda i: (0, 0)),
            pl.BlockSpec((n_sub * k, d), lambda i: (0, 0)),
            pl.BlockSpec((d, HIDDEN), lambda i: (0, 0)),
            pl.BlockSpec((1, HIDDEN), lambda i: (0, 0)),
            pl.BlockSpec((HIDDEN, CPAD), lambda i: (0, 0)),
            pl.BlockSpec((1, CPAD), lambda i: (0, 0)),
        ],
        out_specs=pl.BlockSpec((TM, CPAD), lambda i: (i, 0)),
        out_shape=jax.ShapeDtypeStruct((bt, CPAD), jnp.float32),
        scratch_shapes=[pltpu.VMEM((TM, n_sub * k), jnp.float32)],
        compiler_params=pltpu.CompilerParams(
            dimension_semantics=("arbitrary",),
        ),
    )(z2, cbT, c_sq, cbs, W1, b1r, W2p, b2p)
    return out[:, :CLASSES].reshape(b, t, CLASSES)


# speedup vs baseline: 8.4162x; 1.8421x over previous
import jax
import jax.numpy as jnp
from jax.experimental import pallas as pl
from jax.experimental.pallas import tpu as pltpu

B, T, D = 32, 576, 256
N_SUB, K, SUB = 8, 256, 32
HIDDEN = 1024
CLASSES = 1000
CPAD = 1024
TM = 512
NK = N_SUB * K


def _body(z_ref, cbtbd_ref, csq_ref, cbs_ref, w1_ref, b1_ref, w2_ref, b2_ref,
          out_ref, onehot_scr):
    zt = z_ref[...]
    dist = jnp.dot(zt, cbtbd_ref[...],
                   preferred_element_type=jnp.float32) + csq_ref[...]
    iota = jax.lax.broadcasted_iota(jnp.int32, (TM, K), 1)
    for n in range(N_SUB):
        code = jnp.argmin(dist[:, n * K:(n + 1) * K], axis=1)
        onehot_scr[:, n * K:(n + 1) * K] = (
            iota == code[:, None]).astype(jnp.bfloat16)
    q = jnp.dot(onehot_scr[...], cbs_ref[...],
                preferred_element_type=jnp.float32)
    h = jnp.maximum(jnp.dot(q.astype(jnp.bfloat16), w1_ref[...],
                            preferred_element_type=jnp.float32)
                    + b1_ref[...], 0.0)
    out_ref[...] = jnp.dot(h.astype(jnp.bfloat16), w2_ref[...],
                           preferred_element_type=jnp.float32) + b2_ref[...]


@jax.jit
def kernel(z, codebook, W1, b1, W2, b2):
    b, t, d = z.shape
    n_sub, k, sub = codebook.shape
    bt = b * t
    z2 = z.reshape(bt, d)
    cbT = codebook.transpose(0, 2, 1)
    cbtbd = jnp.concatenate(
        [jnp.pad(-2.0 * cbT[n], ((0, 0), (n * k, (n_sub - 1 - n) * k)))
         for n in range(n_sub)], axis=0)
    csq = jnp.sum(codebook * codebook, axis=-1).reshape(1, n_sub * k)
    cbs = jnp.concatenate(
        [jnp.pad(codebook[n], ((0, 0), (n * sub, d - (n + 1) * sub)))
         for n in range(n_sub)], axis=0).astype(jnp.bfloat16)
    W1b = W1.astype(jnp.bfloat16)
    W2b = jnp.pad(W2, ((0, 0), (0, CPAD - CLASSES))).astype(jnp.bfloat16)
    b2p = jnp.pad(b2, (0, CPAD - CLASSES)).reshape(1, CPAD)
    b1r = b1.reshape(1, HIDDEN)

    grid = (bt // TM,)
    out = pl.pallas_call(
        _body,
        grid=grid,
        in_specs=[
            pl.BlockSpec((TM, d), lambda i: (i, 0)),
            pl.BlockSpec((d, NK), lambda i: (0, 0)),
            pl.BlockSpec((1, NK), lambda i: (0, 0)),
            pl.BlockSpec((NK, d), lambda i: (0, 0)),
            pl.BlockSpec((d, HIDDEN), lambda i: (0, 0)),
            pl.BlockSpec((1, HIDDEN), lambda i: (0, 0)),
            pl.BlockSpec((HIDDEN, CPAD), lambda i: (0, 0)),
            pl.BlockSpec((1, CPAD), lambda i: (0, 0)),
        ],
        out_specs=pl.BlockSpec((TM, CPAD), lambda i: (i, 0)),
        out_shape=jax.ShapeDtypeStruct((bt, CPAD), jnp.float32),
        scratch_shapes=[pltpu.VMEM((TM, NK), jnp.bfloat16)],
        compiler_params=pltpu.CompilerParams(
            dimension_semantics=("arbitrary",),
        ),
    )(z2, cbtbd, csq, cbs, W1b, b1r, W2b, b2p)
    return out[:, :CLASSES].reshape(b, t, CLASSES)


# speedup vs baseline: 10.0684x; 1.1963x over previous
import jax
import jax.numpy as jnp
from jax.experimental import pallas as pl
from jax.experimental.pallas import tpu as pltpu

B, T, D = 32, 576, 256
N_SUB, K, SUB = 8, 256, 32
HIDDEN = 1024
CLASSES = 1000
CPAD = 1024
TM = 2048
HALF = 512
NK = N_SUB * K


def _body(z_ref, cbtbd_ref, csq_ref, cbs_ref, w1_ref, b1_ref, w2_ref, b2_ref,
          out_ref, onehot_scr):
    iota = jax.lax.broadcasted_iota(jnp.int32, (HALF, K), 1)
    for h0 in range(0, TM, HALF):
        rows = pl.ds(h0, HALF)
        zt = z_ref[rows, :]
        dist = jnp.dot(zt, cbtbd_ref[...],
                       preferred_element_type=jnp.float32) + csq_ref[...]
        for n in range(N_SUB):
            code = jnp.argmin(dist[:, n * K:(n + 1) * K], axis=1)
            onehot_scr[rows, n * K:(n + 1) * K] = (
                iota == code[:, None]).astype(jnp.bfloat16)
        q = jnp.dot(onehot_scr[rows, :], cbs_ref[...],
                    preferred_element_type=jnp.float32)
        h = jnp.maximum(jnp.dot(q.astype(jnp.bfloat16), w1_ref[...],
                                preferred_element_type=jnp.float32)
                        + b1_ref[...], 0.0)
        out_ref[rows, :] = jnp.dot(h.astype(jnp.bfloat16), w2_ref[...],
                                   preferred_element_type=jnp.float32) \
            + b2_ref[...]


@jax.jit
def kernel(z, codebook, W1, b1, W2, b2):
    b, t, d = z.shape
    n_sub, k, sub = codebook.shape
    bt = b * t
    z2 = z.reshape(bt, d)
    cbT = codebook.transpose(0, 2, 1)
    cbtbd = jnp.concatenate(
        [jnp.pad(-2.0 * cbT[n], ((0, 0), (n * k, (n_sub - 1 - n) * k)))
         for n in range(n_sub)], axis=0)
    csq = jnp.sum(codebook * codebook, axis=-1).reshape(1, n_sub * k)
    cbs = jnp.concatenate(
        [jnp.pad(codebook[n], ((0, 0), (n * sub, d - (n + 1) * sub)))
         for n in range(n_sub)], axis=0).astype(jnp.bfloat16)
    W1b = W1.astype(jnp.bfloat16)
    W2b = jnp.pad(W2, ((0, 0), (0, CPAD - CLASSES))).astype(jnp.bfloat16)
    b2p = jnp.pad(b2, (0, CPAD - CLASSES)).reshape(1, CPAD)
    b1r = b1.reshape(1, HIDDEN)

    grid = (bt // TM,)
    out = pl.pallas_call(
        _body,
        grid=grid,
        in_specs=[
            pl.BlockSpec((TM, d), lambda i: (i, 0)),
            pl.BlockSpec((d, NK), lambda i: (0, 0)),
            pl.BlockSpec((1, NK), lambda i: (0, 0)),
            pl.BlockSpec((NK, d), lambda i: (0, 0)),
            pl.BlockSpec((d, HIDDEN), lambda i: (0, 0)),
            pl.BlockSpec((1, HIDDEN), lambda i: (0, 0)),
            pl.BlockSpec((HIDDEN, CPAD), lambda i: (0, 0)),
            pl.BlockSpec((1, CPAD), lambda i: (0, 0)),
        ],
        out_specs=pl.BlockSpec((TM, CPAD), lambda i: (i, 0)),
        out_shape=jax.ShapeDtypeStruct((bt, CPAD), jnp.float32),
        scratch_shapes=[pltpu.VMEM((TM, NK), jnp.bfloat16)],
        compiler_params=pltpu.CompilerParams(
            dimension_semantics=("arbitrary",),
        ),
    )(z2, cbtbd, csq, cbs, W1b, b1r, W2b, b2p)
    return out[:, :CLASSES].reshape(b, t, CLASSES)


# speedup vs baseline: 10.1262x; 1.0057x over previous
import jax
import jax.numpy as jnp
from jax.experimental import pallas as pl
from jax.experimental.pallas import tpu as pltpu

B, T, D = 32, 576, 256
N_SUB, K, SUB = 8, 256, 32
HIDDEN = 1024
CLASSES = 1000
TM = 2048
HALF = 512
NK = N_SUB * K


def _body(z_ref, cbtbd_ref, csq_ref, cbs_ref, w1_ref, b1_ref, w2_ref, b2_ref,
          out_ref, onehot_scr):
    iota = jax.lax.broadcasted_iota(jnp.int32, (HALF, K), 1)
    for h0 in range(0, TM, HALF):
        rows = pl.ds(h0, HALF)
        zt = z_ref[rows, :]
        dist = jnp.dot(zt, cbtbd_ref[...],
                       preferred_element_type=jnp.float32) + csq_ref[...]
        for n in range(N_SUB):
            code = jnp.argmin(dist[:, n * K:(n + 1) * K], axis=1)
            onehot_scr[rows, n * K:(n + 1) * K] = (
                iota == code[:, None]).astype(jnp.bfloat16)
        q = jnp.dot(onehot_scr[rows, :], cbs_ref[...],
                    preferred_element_type=jnp.float32)
        h = jnp.maximum(jnp.dot(q.astype(jnp.bfloat16), w1_ref[...],
                                preferred_element_type=jnp.float32)
                        + b1_ref[...], 0.0)
        out_ref[rows, :] = jnp.dot(h.astype(jnp.bfloat16), w2_ref[...],
                                   preferred_element_type=jnp.float32) \
            + b2_ref[...]


@jax.jit
def kernel(z, codebook, W1, b1, W2, b2):
    b, t, d = z.shape
    n_sub, k, sub = codebook.shape
    bt = b * t
    z2 = z.reshape(bt, d)
    cbT = codebook.transpose(0, 2, 1)
    cbtbd = jnp.concatenate(
        [jnp.pad(-2.0 * cbT[n], ((0, 0), (n * k, (n_sub - 1 - n) * k)))
         for n in range(n_sub)], axis=0)
    csq = jnp.sum(codebook * codebook, axis=-1).reshape(1, n_sub * k)
    cbs = jnp.concatenate(
        [jnp.pad(codebook[n], ((0, 0), (n * sub, d - (n + 1) * sub)))
         for n in range(n_sub)], axis=0).astype(jnp.bfloat16)
    W1b = W1.astype(jnp.bfloat16)
    W2b = W2.astype(jnp.bfloat16)
    b2r = b2.reshape(1, CLASSES)
    b1r = b1.reshape(1, HIDDEN)

    grid = (bt // TM,)
    out = pl.pallas_call(
        _body,
        grid=grid,
        in_specs=[
            pl.BlockSpec((TM, d), lambda i: (i, 0)),
            pl.BlockSpec((d, NK), lambda i: (0, 0)),
            pl.BlockSpec((1, NK), lambda i: (0, 0)),
            pl.BlockSpec((NK, d), lambda i: (0, 0)),
            pl.BlockSpec((d, HIDDEN), lambda i: (0, 0)),
            pl.BlockSpec((1, HIDDEN), lambda i: (0, 0)),
            pl.BlockSpec((HIDDEN, CLASSES), lambda i: (0, 0)),
            pl.BlockSpec((1, CLASSES), lambda i: (0, 0)),
        ],
        out_specs=pl.BlockSpec((TM, CLASSES), lambda i: (i, 0)),
        out_shape=jax.ShapeDtypeStruct((bt, CLASSES), jnp.float32),
        scratch_shapes=[pltpu.VMEM((TM, NK), jnp.bfloat16)],
        compiler_params=pltpu.CompilerParams(
            dimension_semantics=("arbitrary",),
        ),
    )(z2, cbtbd, csq, cbs, W1b, b1r, W2b, b2r)
    return out.reshape(b, t, CLASSES)


# speedup vs baseline: 10.1481x; 1.0022x over previous
import jax
import jax.numpy as jnp
from jax.experimental import pallas as pl
from jax.experimental.pallas import tpu as pltpu

B, T, D = 32, 576, 256
N_SUB, K, SUB = 8, 256, 32
HIDDEN = 1024
CLASSES = 1000
TM = 2048
HALF = 512
NK = N_SUB * K


def _body(z_ref, cbtbd_ref, csq_ref, cbs_ref, w1_ref, b1_ref, w2_ref, b2_ref,
          out_ref, onehot_scr):
    iota = jax.lax.broadcasted_iota(jnp.int32, (HALF, K), 1)
    for h0 in range(0, TM, HALF):
        rows = pl.ds(h0, HALF)
        zt = z_ref[rows, :]
        for n in range(N_SUB):
            dist = jnp.dot(zt, cbtbd_ref[:, n * K:(n + 1) * K],
                           preferred_element_type=jnp.float32) \
                + csq_ref[:, n * K:(n + 1) * K]
            code = jnp.argmin(dist, axis=1)
            onehot_scr[rows, n * K:(n + 1) * K] = (
                iota == code[:, None]).astype(jnp.bfloat16)
        q = jnp.dot(onehot_scr[rows, :], cbs_ref[...],
                    preferred_element_type=jnp.float32)
        h = jnp.maximum(jnp.dot(q.astype(jnp.bfloat16), w1_ref[...],
                                preferred_element_type=jnp.float32)
                        + b1_ref[...], 0.0)
        out_ref[rows, :] = jnp.dot(h.astype(jnp.bfloat16), w2_ref[...],
                                   preferred_element_type=jnp.float32) \
            + b2_ref[...]


@jax.jit
def kernel(z, codebook, W1, b1, W2, b2):
    b, t, d = z.shape
    n_sub, k, sub = codebook.shape
    bt = b * t
    z2 = z.reshape(bt, d)
    cbT = codebook.transpose(0, 2, 1)
    cbtbd = jnp.concatenate(
        [jnp.pad(-2.0 * cbT[n], ((0, 0), (n * k, (n_sub - 1 - n) * k)))
         for n in range(n_sub)], axis=0)
    csq = jnp.sum(codebook * codebook, axis=-1).reshape(1, n_sub * k)
    cbs = jnp.concatenate(
        [jnp.pad(codebook[n], ((0, 0), (n * sub, d - (n + 1) * sub)))
         for n in range(n_sub)], axis=0).astype(jnp.bfloat16)
    W1b = W1.astype(jnp.bfloat16)
    W2b = W2.astype(jnp.bfloat16)
    b2r = b2.reshape(1, CLASSES)
    b1r = b1.reshape(1, HIDDEN)

    grid = (bt // TM,)
    out = pl.pallas_call(
        _body,
        grid=grid,
        in_specs=[
            pl.BlockSpec((TM, d), lambda i: (i, 0)),
            pl.BlockSpec((d, NK), lambda i: (0, 0)),
            pl.BlockSpec((1, NK), lambda i: (0, 0)),
            pl.BlockSpec((NK, d), lambda i: (0, 0)),
            pl.BlockSpec((d, HIDDEN), lambda i: (0, 0)),
            pl.BlockSpec((1, HIDDEN), lambda i: (0, 0)),
            pl.BlockSpec((HIDDEN, CLASSES), lambda i: (0, 0)),
            pl.BlockSpec((1, CLASSES), lambda i: (0, 0)),
        ],
        out_specs=pl.BlockSpec((TM, CLASSES), lambda i: (i, 0)),
        out_shape=jax.ShapeDtypeStruct((bt, CLASSES), jnp.float32),
        scratch_shapes=[pltpu.VMEM((TM, NK), jnp.bfloat16)],
        compiler_params=pltpu.CompilerParams(
            dimension_semantics=("arbitrary",),
        ),
    )(z2, cbtbd, csq, cbs, W1b, b1r, W2b, b2r)
    return out.reshape(b, t, CLASSES)
